# baseline (speedup 1.0000x reference)
import jax
import jax.numpy as jnp
from jax import lax
from jax.experimental import pallas as pl
from jax.experimental.pallas import tpu as pltpu
from jax.experimental.pallas import tpu_sc as plsc
import jax.experimental.layout as jex_layout

BATCH = 4096
HIST = 50
DIM = 60
VOCAB = 1000000
NW = 32
SPW = BATCH // NW
TPB = 2 * HIST
KB = SPW // 2
NBUF = 4
LIST = 512
UPB = 5
BUFW = LIST * 16
IDXPAD = 512
OFFS = (0, 16, 32, 44)
INV_LEN = 1.0 / HIST


def _body(idx_hbm, units_hbm, out_hbm, idx_v, idxl_v, bufs, stage, sems):
  wid = lax.axis_index("s") * 2 + lax.axis_index("c")

  pltpu.sync_copy(idx_hbm.at[wid], idx_v.at[pl.ds(0, KB * TPB)])

  iota = lax.iota(jnp.int32, 16)
  pos5 = iota * 5

  tail_mask = iota < (LIST - UPB * TPB)
  zeros_i = jnp.zeros((16,), jnp.int32)
  for b in range(NBUF):
    plsc.store_scatter(idxl_v.at[b], [iota + UPB * TPB], zeros_i,
                       mask=tail_mask)

  def build_list(b, blk):
    base_tok = blk * TPB
    for s in range(2):
      for g in range(4):
        nvalid = 16 if g < 3 else 2
        toff = s * HIST + g * 16
        v = idx_v[pl.ds(base_tok + toff, 16)]
        u0 = lax.shift_right_logical(v * 15, 2)
        pbase = pos5 + UPB * toff
        if nvalid == 16:
          for k in range(UPB):
            plsc.store_scatter(idxl_v.at[b], [pbase + k], u0 + k)
        else:
          m = iota < nvalid
          for k in range(UPB):
            plsc.store_scatter(idxl_v.at[b], [pbase + k], u0 + k, mask=m)

  def issue(b, blk):
    pltpu.async_copy(units_hbm.at[idxl_v.at[b]],
                     bufs.at[pl.ds(b * LIST, LIST)], sems.at[b])

  for b in range(NBUF):
    build_list(b, b)
    issue(b, b)

  def block_step(blk, _):
    b = lax.rem(blk, NBUF)
    pltpu.make_async_copy(
        units_hbm.at[idxl_v.at[b]],
        bufs.at[pl.ds(b * LIST, LIST)], sems.at[b]).wait()

    nxt = blk + NBUF
    @pl.when(nxt < KB)
    def _():
      build_list(b, nxt)
      issue(b, nxt)

    slot0 = b * (LIST * 16)
    base_tok = blk * TPB
    for s in range(2):
      accs = [jnp.zeros((16,), jnp.float32) for _ in OFFS]
      for g in range(4):
        nvalid = 16 if g < 3 else 2
        toff = s * HIST + g * 16
        v = idx_v[pl.ds(base_tok + toff, 16)]
        for t in range(nvalid):
          i_tok = v[t]
          st = (i_tok * 60) & 15
          w0 = (slot0 + (toff + t) * (UPB * 16) + st) + iota
          row0 = lax.shift_right_logical(w0, 4)
          col0 = w0 & 15
          for j, off in enumerate(OFFS):
            if off % 16 == 0:
              x = plsc.load_gather(bufs, [row0 + (off // 16), col0])
            else:
              wj = w0 + off
              x = plsc.load_gather(
                  bufs, [lax.shift_right_logical(wj, 4), wj & 15])
            accs[j] = accs[j] + x
      orow = blk * 2 + s
      for j, off in enumerate(OFFS):
        stage[orow, pl.ds(off, 16)] = accs[j] * INV_LEN
    return 0

  lax.fori_loop(0, KB, block_step, 0)

  pltpu.sync_copy(stage, out_hbm.at[pl.ds(wid * SPW, SPW)])


@jax.jit
def _embed(idx, units):
  mesh = plsc.VectorSubcoreMesh(core_axis_name="c", subcore_axis_name="s",
                                num_cores=2, num_subcores=16)
  scratch = [
      pltpu.VMEM((KB * TPB + IDXPAD,), jnp.int32),
      pltpu.VMEM((NBUF, LIST), jnp.int32),
      pltpu.VMEM((NBUF * LIST, 16), jnp.float32),
      pltpu.VMEM((SPW, DIM), jnp.float32),
      pltpu.SemaphoreType.DMA((NBUF,)),
  ]
  fn = pl.kernel(
      _body,
      out_type=jax.ShapeDtypeStruct((BATCH, DIM), jnp.float32),
      mesh=mesh,
      scratch_types=scratch,
      compiler_params=pltpu.CompilerParams(use_tc_tiling_on_sc=False,
                                           needs_layout_passes=False),
  )
  return fn(idx, units)


def kernel(indices, table):
  idx = indices.astype(jnp.int32).reshape(NW, KB * TPB)
  t_lin = jex_layout.with_layout_constraint(
      table, jex_layout.Layout((0, 1), tiling=()))
  units = t_lin.reshape(VOCAB * DIM // 16, 16)
  return _embed(idx, units)


# speedup vs baseline: 1.0802x; 1.0802x over previous
import jax
import jax.numpy as jnp
from jax import lax
from jax.experimental import pallas as pl
from jax.experimental.pallas import tpu as pltpu
from jax.experimental.pallas import tpu_sc as plsc

BATCH = 4096
HIST = 50
DIM = 60
VOCAB = 1000000
NW = 32
SPW = BATCH // NW
TPB = 2 * HIST
KB = SPW // 2
NBUF = 3
UPB = 2
LIST = UPB * TPB
UNITW = 128
OFFS = (0, 16, 32, 44)
INV_LEN = 1.0 / HIST


def _body(idx_hbm, units_hbm, out_hbm, idx_v, idxl_v, bufs, stage, sems):
  wid = lax.axis_index("s") * 2 + lax.axis_index("c")

  pltpu.sync_copy(idx_hbm.at[wid], idx_v)

  iota = lax.iota(jnp.int32, 16)
  pos2 = iota * UPB

  def build_list(b, blk):
    base_tok = blk * TPB
    for s in range(2):
      for g in range(4):
        nvalid = 16 if g < 3 else 2
        toff = s * HIST + g * 16
        v = idx_v[pl.ds(base_tok + toff, 16)]
        u0 = lax.shift_right_logical(v * DIM, 7)
        pbase = pos2 + UPB * toff
        m = None if nvalid == 16 else (iota < nvalid)
        for k in range(UPB):
          plsc.store_scatter(idxl_v.at[b], [pbase + k], u0 + k, mask=m)

  def issue(b, blk):
    pltpu.async_copy(units_hbm.at[idxl_v.at[b]],
                     bufs.at[pl.ds(b * LIST, LIST)], sems.at[b])

  for b in range(NBUF):
    build_list(b, b)
    issue(b, b)

  def block_step(blk, _):
    b = lax.rem(blk, NBUF)
    pltpu.make_async_copy(
        units_hbm.at[idxl_v.at[b]],
        bufs.at[pl.ds(b * LIST, LIST)], sems.at[b]).wait()

    nxt = blk + NBUF
    @pl.when(nxt < KB)
    def _():
      build_list(b, nxt)
      issue(b, nxt)

    slot0 = b * (LIST * UNITW)
    base_tok = blk * TPB
    for s in range(2):
      accs = [jnp.zeros((16,), jnp.float32) for _ in OFFS]
      for g in range(4):
        nvalid = 16 if g < 3 else 2
        toff = s * HIST + g * 16
        v = idx_v[pl.ds(base_tok + toff, 16)]
        for t in range(nvalid):
          i_tok = v[t]
          st = (i_tok * DIM) & (UNITW - 1)
          w0 = (slot0 + (toff + t) * (UPB * UNITW) + st) + iota
          for j, off in enumerate(OFFS):
            wj = w0 + off
            x = plsc.load_gather(
                bufs,
                [lax.shift_right_logical(wj, 7), wj & (UNITW - 1)])
            accs[j] = accs[j] + x
      orow = blk * 2 + s
      for j, off in enumerate(OFFS):
        stage[orow, pl.ds(off, 16)] = accs[j] * INV_LEN
    return 0

  lax.fori_loop(0, KB, block_step, 0)

  pltpu.sync_copy(stage, out_hbm.at[pl.ds(wid * SPW, SPW)])


@jax.jit
def _embed(idx, units):
  mesh = plsc.VectorSubcoreMesh(core_axis_name="c", subcore_axis_name="s",
                                num_cores=2, num_subcores=16)
  scratch = [
      pltpu.VMEM((KB * TPB,), jnp.int32),
      pltpu.VMEM((NBUF, LIST), jnp.int32),
      pltpu.VMEM((NBUF * LIST, UNITW), jnp.float32),
      pltpu.VMEM((SPW, DIM), jnp.float32),
      pltpu.SemaphoreType.DMA((NBUF,)),
  ]
  fn = pl.kernel(
      _body,
      out_type=jax.ShapeDtypeStruct((BATCH, DIM), jnp.float32),
      mesh=mesh,
      scratch_types=scratch,
      compiler_params=pltpu.CompilerParams(use_tc_tiling_on_sc=False,
                                           needs_layout_passes=False),
  )
  return fn(idx, units)


def kernel(indices, table):
  idx = indices.astype(jnp.int32).reshape(NW, KB * TPB)
  units = table.reshape(VOCAB * DIM // UNITW, UNITW)
  return _embed(idx, units)


# speedup vs baseline: 1.0826x; 1.0022x over previous
import jax
import jax.numpy as jnp
from jax import lax
from jax.experimental import pallas as pl
from jax.experimental.pallas import tpu as pltpu
from jax.experimental.pallas import tpu_sc as plsc
import jax.experimental.layout as jex_layout

BATCH = 4096
HIST = 50
DIM = 60
VOCAB = 1000000
NW = 32
SPW = BATCH // NW
TPB = 2 * HIST
KB = SPW // 2
NBUF = 3
UPB = 2
LIST = UPB * TPB
UNITW = 128
OFFS = (0, 16, 32, 44)
INV_LEN = 1.0 / HIST


def _body(idx_hbm, units_hbm, out_hbm, idx_v, idxl_v, bufs, stage, sems):
  wid = lax.axis_index("s") * 2 + lax.axis_index("c")

  pltpu.sync_copy(idx_hbm.at[wid], idx_v)

  iota = lax.iota(jnp.int32, 16)
  pos2 = iota * UPB

  def build_list(b, blk):
    base_tok = blk * TPB
    for s in range(2):
      for g in range(4):
        nvalid = 16 if g < 3 else 2
        toff = s * HIST + g * 16
        v = idx_v[pl.ds(base_tok + toff, 16)]
        u0 = lax.shift_right_logical(v * DIM, 7)
        pbase = pos2 + UPB * toff
        m = None if nvalid == 16 else (iota < nvalid)
        for k in range(UPB):
          plsc.store_scatter(idxl_v.at[b], [pbase + k], u0 + k, mask=m)

  def issue(b, blk):
    pltpu.async_copy(units_hbm.at[idxl_v.at[b]],
                     bufs.at[pl.ds(b * LIST, LIST)], sems.at[b])

  for b in range(NBUF):
    build_list(b, b)
    issue(b, b)

  def block_step(blk, _):
    b = lax.rem(blk, NBUF)
    pltpu.make_async_copy(
        units_hbm.at[idxl_v.at[b]],
        bufs.at[pl.ds(b * LIST, LIST)], sems.at[b]).wait()

    nxt = blk + NBUF
    @pl.when(nxt < KB)
    def _():
      build_list(b, nxt)
      issue(b, nxt)

    slot0 = b * (LIST * UNITW)
    base_tok = blk * TPB
    for s in range(2):
      accs = [jnp.zeros((16,), jnp.float32) for _ in OFFS]
      for g in range(4):
        nvalid = 16 if g < 3 else 2
        toff = s * HIST + g * 16
        v = idx_v[pl.ds(base_tok + toff, 16)]
        for t in range(nvalid):
          i_tok = v[t]
          st = (i_tok * DIM) & (UNITW - 1)
          w0 = (slot0 + (toff + t) * (UPB * UNITW) + st) + iota
          for j, off in enumerate(OFFS):
            wj = w0 + off
            x = plsc.load_gather(
                bufs,
                [lax.shift_right_logical(wj, 7), wj & (UNITW - 1)])
            accs[j] = accs[j] + x
      orow = blk * 2 + s
      for j, off in enumerate(OFFS):
        stage[orow, pl.ds(off, 16)] = accs[j] * INV_LEN
    return 0

  lax.fori_loop(0, KB, block_step, 0)

  pltpu.sync_copy(stage, out_hbm.at[pl.ds(wid * SPW, SPW)])


@jax.jit
def _embed(idx, units):
  mesh = plsc.VectorSubcoreMesh(core_axis_name="c", subcore_axis_name="s",
                                num_cores=2, num_subcores=16)
  scratch = [
      pltpu.VMEM((KB * TPB,), jnp.int32),
      pltpu.VMEM((NBUF, LIST), jnp.int32),
      pltpu.VMEM((NBUF * LIST, UNITW), jnp.float32),
      pltpu.VMEM((SPW, DIM), jnp.float32),
      pltpu.SemaphoreType.DMA((NBUF,)),
  ]
  fn = pl.kernel(
      _body,
      out_type=jax.ShapeDtypeStruct((BATCH, DIM), jnp.float32),
      mesh=mesh,
      scratch_types=scratch,
      compiler_params=pltpu.CompilerParams(use_tc_tiling_on_sc=False,
                                           needs_layout_passes=False),
  )
  return fn(idx, units)


def kernel(indices, table):
  idx = indices.astype(jnp.int32).reshape(NW, KB * TPB)
  t_lin = jex_layout.with_layout_constraint(
      table, jex_layout.Layout((0, 1), tiling=()))
  units = t_lin.reshape(VOCAB * DIM // UNITW, UNITW)
  return _embed(idx, units)
